# baseline (device time: 224721 ns/iter reference)
import functools

import jax
import jax.numpy as jnp
from jax import lax
from jax.experimental import pallas as pl
from jax.experimental.pallas import tpu as pltpu


def kernel(x):
    m, n = x.shape
    half = m // 2

    sizes = [128, 128, 256] + [512] * 14 + [256, 128, 128]
    assert sum(sizes) == half
    offs = [sum(sizes[:k]) for k in range(len(sizes))]
    n_chunks = len(sizes)
    max_rows = max(sizes)

    def body(x_ref, out_ref, stage_ref, send_ref, acc_ref, recv_ref,
             load_sems, send_sems_a, recv_sems_a,
             send_sems_b, recv_sems_b, copy_sems, copy2_sems):
        my_x = lax.axis_index("x")
        my_y = lax.axis_index("y")
        x_nbr = (1 - my_x, my_y)
        y_nbr = (my_x, 1 - my_y)

        barrier = pltpu.get_barrier_semaphore()
        for nbr in (x_nbr, y_nbr):
            pl.semaphore_signal(
                barrier, inc=1, device_id=nbr,
                device_id_type=pl.DeviceIdType.MESH,
            )
        pl.semaphore_wait(barrier, 2)

        row0 = my_y * half
        other0 = (1 - my_y) * half

        def chunk(k):
            return pl.ds(offs[k], sizes[k])

        def full_chunk(k):
            return pl.ds(row0 + offs[k], sizes[k])

        def other_chunk(k):
            return pl.ds(other0 + offs[k], sizes[k])

        def load(k):
            return pltpu.make_async_copy(
                x_ref.at[full_chunk(k), :],
                stage_ref.at[k % 2, pl.ds(0, sizes[k])],
                load_sems.at[k],
            )

        load(0).start()
        load(1).start()
        rdmas_a = []
        for k in range(n_chunks):
            load(k).wait()
            send_ref[chunk(k), :] = (
                stage_ref[k % 2, pl.ds(0, sizes[k])].astype(jnp.bfloat16)
            )
            rdma_a = pltpu.make_async_remote_copy(
                src_ref=send_ref.at[chunk(k), :],
                dst_ref=acc_ref.at[chunk(k), :],
                send_sem=send_sems_a.at[k],
                recv_sem=recv_sems_a.at[k],
                device_id=x_nbr,
                device_id_type=pl.DeviceIdType.MESH,
            )
            rdma_a.start()
            rdmas_a.append(rdma_a)
            if k + 2 < n_chunks:
                load(k + 2).start()

        rdmas_b, locals_ = [], []
        for k in range(n_chunks):
            rdmas_a[k].wait_recv()
            acc_ref[chunk(k), :] = acc_ref[chunk(k), :] + send_ref[chunk(k), :]
            rdma_b = pltpu.make_async_remote_copy(
                src_ref=acc_ref.at[chunk(k), :],
                dst_ref=recv_ref.at[chunk(k), :],
                send_sem=send_sems_b.at[k],
                recv_sem=recv_sems_b.at[k],
                device_id=y_nbr,
                device_id_type=pl.DeviceIdType.MESH,
            )
            rdma_b.start()
            rdmas_b.append(rdma_b)
            local = pltpu.make_async_copy(
                acc_ref.at[chunk(k), :], out_ref.at[full_chunk(k), :],
                copy_sems.at[k],
            )
            local.start()
            locals_.append(local)

        locals2 = []
        for k in range(n_chunks):
            rdmas_b[k].wait_recv()
            local2 = pltpu.make_async_copy(
                recv_ref.at[chunk(k), :], out_ref.at[other_chunk(k), :],
                copy2_sems.at[k],
            )
            local2.start()
            locals2.append(local2)
        for k in range(n_chunks):
            rdmas_b[k].wait_send()
            rdmas_a[k].wait_send()
            locals_[k].wait()
            locals2[k].wait()

        @functools.partial(
            pl.run_scoped, second_barrier=pltpu.SemaphoreType.REGULAR
        )
        def _(second_barrier):
            for nbr in (x_nbr, y_nbr):
                pl.semaphore_signal(
                    second_barrier, inc=1, device_id=nbr,
                    device_id_type=pl.DeviceIdType.MESH,
                )
            pl.semaphore_wait(second_barrier, 2)

    return pl.pallas_call(
        body,
        out_shape=jax.ShapeDtypeStruct((m, n), jnp.bfloat16),
        in_specs=[pl.BlockSpec(memory_space=pl.ANY)],
        out_specs=pl.BlockSpec(memory_space=pl.ANY),
        scratch_shapes=[
            pltpu.VMEM((2, max_rows, n), jnp.float32),
            pltpu.VMEM((half, n), jnp.bfloat16),
            pltpu.VMEM((half, n), jnp.bfloat16),
            pltpu.VMEM((half, n), jnp.bfloat16),
            pltpu.SemaphoreType.DMA((n_chunks,)),
            pltpu.SemaphoreType.DMA((n_chunks,)),
            pltpu.SemaphoreType.DMA((n_chunks,)),
            pltpu.SemaphoreType.DMA((n_chunks,)),
            pltpu.SemaphoreType.DMA((n_chunks,)),
            pltpu.SemaphoreType.DMA((n_chunks,)),
            pltpu.SemaphoreType.DMA((n_chunks,)),
        ],
        compiler_params=pltpu.CompilerParams(
            collective_id=0, vmem_limit_bytes=60 * 1024 * 1024,
        ),
    )(x)
